# Initial kernel scaffold; baseline (speedup 1.0000x reference)
import jax
import jax.numpy as jnp
from jax.experimental import pallas as pl


def kernel(positions, edge_index, bond_types, batch):
    raise NotImplementedError("write your pallas kernel here")


# speedup vs baseline: 81.1864x; 81.1864x over previous
import functools
import jax
import jax.numpy as jnp
from jax import lax
from jax.experimental import pallas as pl
from jax.experimental.pallas import tpu as pltpu
from jax.experimental.pallas import tpu_sc as plsc

N_NODES_ = 100000
N_EDGES_ = 3200000
NC = 2
NS = 16
NW = NC * NS
E_PER_W = N_EDGES_ // NW
CHUNK = 4000
N_CHUNKS = E_PER_W // CHUNK
GROUPS = CHUNK // 16


def _bond_loss_sc(row_hbm, col_hbm, bond_hbm, px_hbm, py_hbm, pz_hbm,
                  out_hbm,
                  tx_sh, ty_sh, tz_sh, idx_r, idx_c, bt_v,
                  rx_v, ry_v, rz_v, cx_v, cy_v, cz_v, acc_v,
                  sem_tab, sem_g):
    cid = lax.axis_index("c")
    sid = lax.axis_index("s")
    wid = sid * NC + cid

    @pl.when(sid == 0)
    def _():
        a = pltpu.async_copy(px_hbm, tx_sh, sem_tab)
        b = pltpu.async_copy(py_hbm, ty_sh, sem_tab)
        c = pltpu.async_copy(pz_hbm, tz_sh, sem_tab)
        a.wait()
        b.wait()
        c.wait()
    plsc.subcore_barrier()

    base_w = wid * E_PER_W
    total = jnp.zeros((16,), jnp.float32)
    for c in range(N_CHUNKS):
        base_e = base_w + c * CHUNK
        pltpu.sync_copy(row_hbm.at[pl.ds(base_e, CHUNK)], idx_r)
        pltpu.sync_copy(col_hbm.at[pl.ds(base_e, CHUNK)], idx_c)
        pltpu.sync_copy(bond_hbm.at[pl.ds(base_e, CHUNK)], bt_v)
        cps = [
            pltpu.async_copy(tx_sh.at[idx_r], rx_v, sem_g),
            pltpu.async_copy(ty_sh.at[idx_r], ry_v, sem_g),
            pltpu.async_copy(tz_sh.at[idx_r], rz_v, sem_g),
            pltpu.async_copy(tx_sh.at[idx_c], cx_v, sem_g),
            pltpu.async_copy(ty_sh.at[idx_c], cy_v, sem_g),
            pltpu.async_copy(tz_sh.at[idx_c], cz_v, sem_g),
        ]
        for cp in cps:
            cp.wait()

        def body(g, acc):
            sl = pl.ds(g * 16, 16)
            dx = rx_v[sl] - cx_v[sl]
            dy = ry_v[sl] - cy_v[sl]
            dz = rz_v[sl] - cz_v[sl]
            s = dx * dx + dy * dy + dz * dz
            i = lax.bitcast_convert_type(s, jnp.int32)
            y = lax.bitcast_convert_type(
                jnp.full((16,), 0x5F3759DF, jnp.int32) - (i >> 1),
                jnp.float32)
            half_s = 0.5 * s
            y = y * (1.5 - half_s * y * y)
            y = y * (1.5 - half_s * y * y)
            y = y * (1.5 - half_s * y * y)
            ln = jnp.where(s > 0.0, s * y, 0.0)
            bt = bt_v[sl]
            exp_len = jnp.full((16,), 1.54, jnp.float32)
            exp_len = jnp.where(bt == 1, 1.34, exp_len)
            exp_len = jnp.where(bt == 2, 1.2, exp_len)
            exp_len = jnp.where(bt == 3, 1.4, exp_len)
            d = ln - exp_len
            return acc + d * d

        total = lax.fori_loop(0, GROUPS, body, total)

    acc_v[...] = total
    pltpu.sync_copy(acc_v, out_hbm.at[wid])


def kernel(positions, edge_index, bond_types, batch):
    del batch
    pos_t = positions.T
    px = pos_t[0]
    py = pos_t[1]
    pz = pos_t[2]
    row = edge_index[0]
    col = edge_index[1]

    mesh = plsc.VectorSubcoreMesh(core_axis_name="c", subcore_axis_name="s")
    partials = pl.kernel(
        _bond_loss_sc,
        mesh=mesh,
        out_type=jax.ShapeDtypeStruct((NW, 16), jnp.float32),
        scratch_types=[
            pltpu.VMEM_SHARED((N_NODES_,), jnp.float32),
            pltpu.VMEM_SHARED((N_NODES_,), jnp.float32),
            pltpu.VMEM_SHARED((N_NODES_,), jnp.float32),
            pltpu.VMEM((CHUNK,), jnp.int32),
            pltpu.VMEM((CHUNK,), jnp.int32),
            pltpu.VMEM((CHUNK,), jnp.int32),
            pltpu.VMEM((CHUNK,), jnp.float32),
            pltpu.VMEM((CHUNK,), jnp.float32),
            pltpu.VMEM((CHUNK,), jnp.float32),
            pltpu.VMEM((CHUNK,), jnp.float32),
            pltpu.VMEM((CHUNK,), jnp.float32),
            pltpu.VMEM((CHUNK,), jnp.float32),
            pltpu.VMEM((16,), jnp.float32),
            pltpu.SemaphoreType.DMA,
            pltpu.SemaphoreType.DMA,
        ],
    )(row, col, bond_types, px, py, pz)
    return jnp.sum(partials) / jnp.float32(N_EDGES_)


# speedup vs baseline: 113.0815x; 1.3929x over previous
import functools
import jax
import jax.numpy as jnp
from jax import lax
from jax.experimental import pallas as pl
from jax.experimental.pallas import tpu as pltpu
from jax.experimental.pallas import tpu_sc as plsc

N_NODES_ = 100000
N_EDGES_ = 3200000
NC = 2
NS = 16
NW = NC * NS
E_PER_W = N_EDGES_ // NW
CHUNK = 4000
N_CHUNKS = E_PER_W // CHUNK
GROUPS = CHUNK // 16


def _bond_loss_sc(row_hbm, col_hbm, bond_hbm, px_hbm, py_hbm, pz_hbm,
                  out_hbm,
                  tx_sh, ty_sh, tz_sh,
                  idx_r0, idx_c0, bt_v0, rx_v0, ry_v0, rz_v0,
                  cx_v0, cy_v0, cz_v0,
                  idx_r1, idx_c1, bt_v1, rx_v1, ry_v1, rz_v1,
                  cx_v1, cy_v1, cz_v1,
                  acc_v,
                  sem_tab, sem_i0, sem_i1, sem_g0, sem_g1):
    cid = lax.axis_index("c")
    sid = lax.axis_index("s")
    wid = sid * NC + cid

    @pl.when(sid == 0)
    def _():
        a = pltpu.async_copy(px_hbm, tx_sh, sem_tab)
        b = pltpu.async_copy(py_hbm, ty_sh, sem_tab)
        c = pltpu.async_copy(pz_hbm, tz_sh, sem_tab)
        a.wait()
        b.wait()
        c.wait()
    plsc.subcore_barrier()

    sets = (
        (idx_r0, idx_c0, bt_v0, rx_v0, ry_v0, rz_v0, cx_v0, cy_v0, cz_v0,
         sem_i0, sem_g0),
        (idx_r1, idx_c1, bt_v1, rx_v1, ry_v1, rz_v1, cx_v1, cy_v1, cz_v1,
         sem_i1, sem_g1),
    )
    base_w = wid * E_PER_W

    def issue_idx(c, s):
        idx_r, idx_c, bt_v = s[0], s[1], s[2]
        sem_i = s[9]
        base_e = base_w + c * CHUNK
        return [
            pltpu.async_copy(row_hbm.at[pl.ds(base_e, CHUNK)], idx_r, sem_i),
            pltpu.async_copy(col_hbm.at[pl.ds(base_e, CHUNK)], idx_c, sem_i),
            pltpu.async_copy(bond_hbm.at[pl.ds(base_e, CHUNK)], bt_v, sem_i),
        ]

    def issue_gathers(s):
        idx_r, idx_c = s[0], s[1]
        rx_v, ry_v, rz_v, cx_v, cy_v, cz_v = s[3:9]
        sem_g = s[10]
        return [
            pltpu.async_copy(tx_sh.at[idx_r], rx_v, sem_g),
            pltpu.async_copy(ty_sh.at[idx_r], ry_v, sem_g),
            pltpu.async_copy(tz_sh.at[idx_r], rz_v, sem_g),
            pltpu.async_copy(tx_sh.at[idx_c], cx_v, sem_g),
            pltpu.async_copy(ty_sh.at[idx_c], cy_v, sem_g),
            pltpu.async_copy(tz_sh.at[idx_c], cz_v, sem_g),
        ]

    def compute(s, acc0):
        bt_v = s[2]
        rx_v, ry_v, rz_v, cx_v, cy_v, cz_v = s[3:9]

        def body(g, acc):
            sl = pl.ds(g * 16, 16)
            dx = rx_v[sl] - cx_v[sl]
            dy = ry_v[sl] - cy_v[sl]
            dz = rz_v[sl] - cz_v[sl]
            s2 = dx * dx + dy * dy + dz * dz
            i = lax.bitcast_convert_type(s2, jnp.int32)
            y = lax.bitcast_convert_type(
                jnp.full((16,), 0x5F3759DF, jnp.int32) - (i >> 1),
                jnp.float32)
            half_s = 0.5 * s2
            y = y * (1.5 - half_s * y * y)
            y = y * (1.5 - half_s * y * y)
            y = y * (1.5 - half_s * y * y)
            ln = jnp.where(s2 > 0.0, s2 * y, 0.0)
            bt = bt_v[sl]
            exp_len = jnp.full((16,), 1.54, jnp.float32)
            exp_len = jnp.where(bt == 1, 1.34, exp_len)
            exp_len = jnp.where(bt == 2, 1.2, exp_len)
            exp_len = jnp.where(bt == 3, 1.4, exp_len)
            d = ln - exp_len
            return acc + d * d

        return lax.fori_loop(0, GROUPS, body, acc0)

    total = jnp.zeros((16,), jnp.float32)
    for cp in issue_idx(0, sets[0]):
        cp.wait()
    gcps = issue_gathers(sets[0])
    for c in range(N_CHUNKS):
        cur = sets[c % 2]
        nxt = sets[(c + 1) % 2]
        if c + 1 < N_CHUNKS:
            icps = issue_idx(c + 1, nxt)
        for cp in gcps:
            cp.wait()
        if c + 1 < N_CHUNKS:
            for cp in icps:
                cp.wait()
            next_gcps = issue_gathers(nxt)
        total = compute(cur, total)
        if c + 1 < N_CHUNKS:
            gcps = next_gcps

    acc_v[...] = total
    pltpu.sync_copy(acc_v, out_hbm.at[wid])


def kernel(positions, edge_index, bond_types, batch):
    del batch
    pos_t = positions.T
    px = pos_t[0]
    py = pos_t[1]
    pz = pos_t[2]
    row = edge_index[0]
    col = edge_index[1]

    chunk_bufs = (
        [pltpu.VMEM((CHUNK,), jnp.int32)] * 3
        + [pltpu.VMEM((CHUNK,), jnp.float32)] * 6
    )
    mesh = plsc.VectorSubcoreMesh(core_axis_name="c", subcore_axis_name="s")
    partials = pl.kernel(
        _bond_loss_sc,
        mesh=mesh,
        out_type=jax.ShapeDtypeStruct((NW, 16), jnp.float32),
        scratch_types=[
            pltpu.VMEM_SHARED((N_NODES_,), jnp.float32),
            pltpu.VMEM_SHARED((N_NODES_,), jnp.float32),
            pltpu.VMEM_SHARED((N_NODES_,), jnp.float32),
        ] + chunk_bufs + chunk_bufs + [
            pltpu.VMEM((16,), jnp.float32),
            pltpu.SemaphoreType.DMA,
            pltpu.SemaphoreType.DMA,
            pltpu.SemaphoreType.DMA,
            pltpu.SemaphoreType.DMA,
            pltpu.SemaphoreType.DMA,
        ],
    )(row, col, bond_types, px, py, pz)
    return jnp.sum(partials) / jnp.float32(N_EDGES_)


# speedup vs baseline: 113.2854x; 1.0018x over previous
import functools
import jax
import jax.numpy as jnp
from jax import lax
from jax.experimental import pallas as pl
from jax.experimental.pallas import tpu as pltpu
from jax.experimental.pallas import tpu_sc as plsc

N_NODES_ = 100000
N_EDGES_ = 3200000
NC = 2
NS = 16
NW = NC * NS
E_PER_W = N_EDGES_ // NW
CHUNK = 4000
N_CHUNKS = E_PER_W // CHUNK
GROUPS = CHUNK // 16


def _bond_loss_sc(row_hbm, col_hbm, bond_hbm, px_hbm, py_hbm, pz_hbm,
                  out_hbm,
                  tx_sh, ty_sh, tz_sh,
                  idx_r0, idx_c0, bt_v0, rx_v0, ry_v0, rz_v0,
                  cx_v0, cy_v0, cz_v0,
                  idx_r1, idx_c1, bt_v1, rx_v1, ry_v1, rz_v1,
                  cx_v1, cy_v1, cz_v1,
                  acc_v,
                  sem_tab, sem_i0, sem_i1, sem_g0, sem_g1):
    cid = lax.axis_index("c")
    sid = lax.axis_index("s")
    wid = sid * NC + cid

    @pl.when(sid == 0)
    def _():
        a = pltpu.async_copy(px_hbm, tx_sh, sem_tab)
        b = pltpu.async_copy(py_hbm, ty_sh, sem_tab)
        c = pltpu.async_copy(pz_hbm, tz_sh, sem_tab)
        a.wait()
        b.wait()
        c.wait()
    plsc.subcore_barrier()

    sets = (
        (idx_r0, idx_c0, bt_v0, rx_v0, ry_v0, rz_v0, cx_v0, cy_v0, cz_v0,
         sem_i0, sem_g0),
        (idx_r1, idx_c1, bt_v1, rx_v1, ry_v1, rz_v1, cx_v1, cy_v1, cz_v1,
         sem_i1, sem_g1),
    )
    base_w = wid * E_PER_W

    def issue_idx(c, s):
        idx_r, idx_c, bt_v = s[0], s[1], s[2]
        sem_i = s[9]
        base_e = base_w + c * CHUNK
        return [
            pltpu.async_copy(row_hbm.at[pl.ds(base_e, CHUNK)], idx_r, sem_i),
            pltpu.async_copy(col_hbm.at[pl.ds(base_e, CHUNK)], idx_c, sem_i),
            pltpu.async_copy(bond_hbm.at[pl.ds(base_e, CHUNK)], bt_v, sem_i),
        ]

    def issue_gathers(s):
        idx_r, idx_c = s[0], s[1]
        rx_v, ry_v, rz_v, cx_v, cy_v, cz_v = s[3:9]
        sem_g = s[10]
        return [
            pltpu.async_copy(tx_sh.at[idx_r], rx_v, sem_g),
            pltpu.async_copy(ty_sh.at[idx_r], ry_v, sem_g),
            pltpu.async_copy(tz_sh.at[idx_r], rz_v, sem_g),
            pltpu.async_copy(tx_sh.at[idx_c], cx_v, sem_g),
            pltpu.async_copy(ty_sh.at[idx_c], cy_v, sem_g),
            pltpu.async_copy(tz_sh.at[idx_c], cz_v, sem_g),
        ]

    def compute(s, acc0):
        bt_v = s[2]
        rx_v, ry_v, rz_v, cx_v, cy_v, cz_v = s[3:9]

        def one(sl):
            dx = rx_v[sl] - cx_v[sl]
            dy = ry_v[sl] - cy_v[sl]
            dz = rz_v[sl] - cz_v[sl]
            s2 = dx * dx + dy * dy + dz * dz
            i = lax.bitcast_convert_type(s2, jnp.int32)
            y = lax.bitcast_convert_type(
                jnp.full((16,), 0x5F3759DF, jnp.int32) - (i >> 1),
                jnp.float32)
            half_s = 0.5 * s2
            y = y * (1.5 - half_s * y * y)
            y = y * (1.5 - half_s * y * y)
            ln = jnp.where(s2 > 0.0, s2 * y, 0.0)
            btf = bt_v[sl].astype(jnp.float32)
            exp_len = ((0.04666667 * btf - 0.11) * btf
                       - 0.13666667) * btf + 1.54
            d = ln - exp_len
            return d * d

        def body(g, acc):
            k = g * 32
            return (acc + one(pl.ds(k, 16))) + one(pl.ds(k + 16, 16))

        return lax.fori_loop(0, GROUPS // 2, body, acc0)

    total = jnp.zeros((16,), jnp.float32)
    for cp in issue_idx(0, sets[0]):
        cp.wait()
    gcps = issue_gathers(sets[0])
    for c in range(N_CHUNKS):
        cur = sets[c % 2]
        nxt = sets[(c + 1) % 2]
        if c + 1 < N_CHUNKS:
            icps = issue_idx(c + 1, nxt)
        for cp in gcps:
            cp.wait()
        if c + 1 < N_CHUNKS:
            for cp in icps:
                cp.wait()
            next_gcps = issue_gathers(nxt)
        total = compute(cur, total)
        if c + 1 < N_CHUNKS:
            gcps = next_gcps

    acc_v[...] = total
    pltpu.sync_copy(acc_v, out_hbm.at[wid])


def kernel(positions, edge_index, bond_types, batch):
    del batch
    pos_t = positions.T
    px = pos_t[0]
    py = pos_t[1]
    pz = pos_t[2]
    row = edge_index[0]
    col = edge_index[1]

    chunk_bufs = (
        [pltpu.VMEM((CHUNK,), jnp.int32)] * 3
        + [pltpu.VMEM((CHUNK,), jnp.float32)] * 6
    )
    mesh = plsc.VectorSubcoreMesh(core_axis_name="c", subcore_axis_name="s")
    partials = pl.kernel(
        _bond_loss_sc,
        mesh=mesh,
        out_type=jax.ShapeDtypeStruct((NW, 16), jnp.float32),
        scratch_types=[
            pltpu.VMEM_SHARED((N_NODES_,), jnp.float32),
            pltpu.VMEM_SHARED((N_NODES_,), jnp.float32),
            pltpu.VMEM_SHARED((N_NODES_,), jnp.float32),
        ] + chunk_bufs + chunk_bufs + [
            pltpu.VMEM((16,), jnp.float32),
            pltpu.SemaphoreType.DMA,
            pltpu.SemaphoreType.DMA,
            pltpu.SemaphoreType.DMA,
            pltpu.SemaphoreType.DMA,
            pltpu.SemaphoreType.DMA,
        ],
    )(row, col, bond_types, px, py, pz)
    return jnp.sum(partials) / jnp.float32(N_EDGES_)
